# Initial kernel scaffold; baseline (speedup 1.0000x reference)
import jax
import jax.numpy as jnp
from jax.experimental import pallas as pl


def kernel(x, edge_index, W1, b1, W2, b2, W3, b3):
    raise NotImplementedError("write your pallas kernel here")


# speedup vs baseline: 9.1059x; 9.1059x over previous
import functools

import jax
import jax.numpy as jnp
from jax import lax
from jax.experimental import pallas as pl
from jax.experimental.pallas import tpu as pltpu
from jax.experimental.pallas import tpu_sc as plsc

N = 10000
E = 320000
D = 128
NPAD = 10240
RB = 512
NBLK = NPAD // RB
NW = 32
CHUNK = 128
EPAD = 327680
EW = EPAD // NW
WCHUNKS = EW // CHUNK
ROWS_PER_TILE = NPAD // 16


def _mesh():
    return plsc.VectorSubcoreMesh(core_axis_name="c", subcore_axis_name="s")


def _sc_deg(dst2d, zeros16):
    @functools.partial(
        pl.kernel,
        mesh=_mesh(),
        out_type=jax.ShapeDtypeStruct((2, NPAD, 16), jnp.float32),
        scratch_types=[
            pltpu.VMEM((WCHUNKS, CHUNK), jnp.int32),
            pltpu.VMEM((CHUNK, 16), jnp.float32),
            pltpu.VMEM_SHARED((NPAD, 16), jnp.float32),
        ],
    )
    def k(dst_h, z_h, deg_h, didx_v, ones_v, deg2_s):
        cid = lax.axis_index("c")
        sid = lax.axis_index("s")
        wid = sid * 2 + cid

        def fill(i, _):
            ones_v[i] = jnp.ones((16,), jnp.float32)
            return 0

        lax.fori_loop(0, CHUNK, fill, 0)
        sl = pl.ds(sid * ROWS_PER_TILE, ROWS_PER_TILE)
        pltpu.sync_copy(z_h.at[sl], deg2_s.at[sl])
        pltpu.sync_copy(dst_h.at[pl.ds(wid * WCHUNKS, WCHUNKS)], didx_v)
        plsc.subcore_barrier()

        def body(c, _):
            pltpu.sync_copy(ones_v, deg2_s.at[didx_v.at[c]], add=True)
            return 0

        lax.fori_loop(0, WCHUNKS, body, 0)
        plsc.subcore_barrier()
        pltpu.sync_copy(deg2_s.at[sl], deg_h.at[cid, sl])

    return k(dst2d, zeros16)


def _sc_scatter(y, src2d, dst2d, zeros128):
    @functools.partial(
        pl.kernel,
        mesh=_mesh(),
        out_type=jax.ShapeDtypeStruct((2, NPAD, D), jnp.float32),
        scratch_types=[
            pltpu.VMEM((WCHUNKS, CHUNK), jnp.int32),
            pltpu.VMEM((WCHUNKS, CHUNK), jnp.int32),
            pltpu.VMEM((CHUNK, D), jnp.float32),
            pltpu.SemaphoreType.DMA,
            pltpu.VMEM_SHARED((NPAD, D), jnp.float32),
        ],
    )
    def k(y_h, src_h, dst_h, z_h, acc_h, sidx_v, didx_v, rows_v, sem, acc_s):
        cid = lax.axis_index("c")
        sid = lax.axis_index("s")
        wid = sid * 2 + cid
        sl = pl.ds(sid * ROWS_PER_TILE, ROWS_PER_TILE)
        pltpu.sync_copy(z_h.at[sl], acc_s.at[sl])
        pltpu.sync_copy(src_h.at[pl.ds(wid * WCHUNKS, WCHUNKS)], sidx_v)
        pltpu.sync_copy(dst_h.at[pl.ds(wid * WCHUNKS, WCHUNKS)], didx_v)
        plsc.subcore_barrier()

        def body(c, _):
            pltpu.async_copy(y_h.at[sidx_v.at[c]], rows_v, sem).wait()
            pltpu.sync_copy(rows_v, acc_s.at[didx_v.at[c]], add=True)
            return 0

        lax.fori_loop(0, WCHUNKS, body, 0)
        plsc.subcore_barrier()
        pltpu.sync_copy(acc_s.at[sl], acc_h.at[cid, sl])

    return k(y, src2d, dst2d, zeros128)


def _dinv_block(deg_block):
    d = deg_block[0] + deg_block[1]
    s = d[:, 0:1] + 1.0
    return lax.rsqrt(s)


def _tc1_body(x_r, w_r, deg_r, y_r):
    dinv = _dinv_block(deg_r[...])
    xw = jnp.dot(x_r[...], w_r[...], preferred_element_type=jnp.float32)
    y_r[...] = xw * dinv


def _tc1(x_pad, W1, degp):
    return pl.pallas_call(
        _tc1_body,
        grid=(NBLK,),
        in_specs=[
            pl.BlockSpec((RB, D), lambda i: (i, 0)),
            pl.BlockSpec((D, D), lambda i: (0, 0)),
            pl.BlockSpec((2, RB, 16), lambda i: (0, i, 0)),
        ],
        out_specs=pl.BlockSpec((RB, D), lambda i: (i, 0)),
        out_shape=jax.ShapeDtypeStruct((NPAD, D), jnp.float32),
    )(x_pad, W1, degp)


def _tc2_body(acc_r, y1_r, deg_r, w_r, b_r, y2_r):
    dinv = _dinv_block(deg_r[...])
    acc = acc_r[...]
    s = acc[0] + acc[1] + y1_r[...]
    h = jnp.maximum(s * dinv + b_r[...], 0.0)
    y2_r[...] = jnp.dot(h, w_r[...], preferred_element_type=jnp.float32) * dinv


def _tc2(acc1, y1, degp, W2, b1):
    return pl.pallas_call(
        _tc2_body,
        grid=(NBLK,),
        in_specs=[
            pl.BlockSpec((2, RB, D), lambda i: (0, i, 0)),
            pl.BlockSpec((RB, D), lambda i: (i, 0)),
            pl.BlockSpec((2, RB, 16), lambda i: (0, i, 0)),
            pl.BlockSpec((D, D), lambda i: (0, 0)),
            pl.BlockSpec((1, D), lambda i: (0, 0)),
        ],
        out_specs=pl.BlockSpec((RB, D), lambda i: (i, 0)),
        out_shape=jax.ShapeDtypeStruct((NPAD, D), jnp.float32),
    )(acc1, y1, degp, W2, b1)


def _tc3_body(acc_r, y2_r, deg_r, b2_r, w3_r, b3_r, o_r):
    i = pl.program_id(0)
    dinv = _dinv_block(deg_r[...])
    acc = acc_r[...]
    s = acc[0] + acc[1] + y2_r[...]
    h2 = jnp.maximum(s * dinv + b2_r[...], 0.0)
    t = jnp.dot(h2, w3_r[...], preferred_element_type=jnp.float32)[:, 0:1]
    v = jnp.maximum(t + b3_r[...], 0.0)
    rows = lax.broadcasted_iota(jnp.int32, (RB, 1), 0) + i * RB
    v = jnp.where(rows < N, v, 0.0)
    ps = jnp.sum(v)

    @pl.when(i == 0)
    def _():
        o_r[...] = jnp.zeros_like(o_r)

    o_r[...] = o_r[...] + ps

    @pl.when(i == NBLK - 1)
    def _():
        o_r[...] = o_r[...] * (1.0 / N)


def _tc3(acc2, y2, degp, b2, W3p, b3):
    return pl.pallas_call(
        _tc3_body,
        grid=(NBLK,),
        in_specs=[
            pl.BlockSpec((2, RB, D), lambda i: (0, i, 0)),
            pl.BlockSpec((RB, D), lambda i: (i, 0)),
            pl.BlockSpec((2, RB, 16), lambda i: (0, i, 0)),
            pl.BlockSpec((1, D), lambda i: (0, 0)),
            pl.BlockSpec((D, D), lambda i: (0, 0)),
            pl.BlockSpec((1, 1), lambda i: (0, 0)),
        ],
        out_specs=pl.BlockSpec((1, 1), lambda i: (0, 0)),
        out_shape=jax.ShapeDtypeStruct((1, 1), jnp.float32),
    )(acc2, y2, degp, b2, W3p, b3)


def kernel(x, edge_index, W1, b1, W2, b2, W3, b3):
    f32 = jnp.float32
    ei = edge_index.astype(jnp.int32)
    pad = jnp.full((EPAD - E,), N, jnp.int32)
    src2d = jnp.concatenate([ei[0], pad]).reshape(EPAD // CHUNK, CHUNK)
    dst2d = jnp.concatenate([ei[1], pad]).reshape(EPAD // CHUNK, CHUNK)
    x_pad = jnp.pad(x.astype(f32), ((0, NPAD - N), (0, 0)))
    zeros16 = jnp.zeros((NPAD, 16), f32)
    zeros128 = jnp.zeros((NPAD, D), f32)
    W3p = jnp.pad(W3.astype(f32), ((0, 0), (0, D - W3.shape[1])))

    degp = _sc_deg(dst2d, zeros16)
    y1 = _tc1(x_pad, W1.astype(f32), degp)
    acc1 = _sc_scatter(y1, src2d, dst2d, zeros128)
    y2 = _tc2(acc1, y1, degp, W2.astype(f32), b1.astype(f32).reshape(1, D))
    acc2 = _sc_scatter(y2, src2d, dst2d, zeros128)
    out = _tc3(acc2, y2, degp, b2.astype(f32).reshape(1, D), W3p,
               b3.astype(f32).reshape(1, 1))
    return out


# speedup vs baseline: 10.3295x; 1.1344x over previous
import functools

import jax
import jax.numpy as jnp
from jax import lax
from jax.experimental import pallas as pl
from jax.experimental.pallas import tpu as pltpu
from jax.experimental.pallas import tpu_sc as plsc

N = 10000
E = 320000
D = 128
NPAD = 10240
RB = 512
NBLK = NPAD // RB
NW = 32
CHUNK = 128
EPAD = 327680
EW = EPAD // NW
WCHUNKS = EW // CHUNK
ROWS_PER_TILE = NPAD // 16
HCHUNKS = WCHUNKS // 2


def _mesh():
    return plsc.VectorSubcoreMesh(core_axis_name="c", subcore_axis_name="s")


def _sc_deg(dst2d, zeros16):
    @functools.partial(
        pl.kernel,
        mesh=_mesh(),
        out_type=jax.ShapeDtypeStruct((2, NPAD, 16), jnp.float32),
        scratch_types=[
            pltpu.VMEM((WCHUNKS, CHUNK), jnp.int32),
            pltpu.VMEM((CHUNK, 16), jnp.float32),
            pltpu.VMEM_SHARED((NPAD, 16), jnp.float32),
        ],
    )
    def k(dst_h, z_h, deg_h, didx_v, ones_v, deg2_s):
        cid = lax.axis_index("c")
        sid = lax.axis_index("s")
        wid = sid * 2 + cid

        def fill(i, _):
            ones_v[i] = jnp.ones((16,), jnp.float32)
            return 0

        lax.fori_loop(0, CHUNK, fill, 0)
        sl = pl.ds(sid * ROWS_PER_TILE, ROWS_PER_TILE)
        pltpu.sync_copy(z_h.at[sl], deg2_s.at[sl])
        pltpu.sync_copy(dst_h.at[pl.ds(wid * WCHUNKS, WCHUNKS)], didx_v)
        plsc.subcore_barrier()

        def body(c, _):
            pltpu.sync_copy(ones_v, deg2_s.at[didx_v.at[c]], add=True)
            return 0

        lax.fori_loop(0, WCHUNKS, body, 0)
        plsc.subcore_barrier()
        pltpu.sync_copy(deg2_s.at[sl], deg_h.at[cid, sl])

    return k(dst2d, zeros16)


def _sc_scatter(y, src2d, dst2d, zeros128):
    @functools.partial(
        pl.kernel,
        mesh=_mesh(),
        out_type=jax.ShapeDtypeStruct((2, NPAD, D), jnp.float32),
        scratch_types=[
            pltpu.VMEM((HCHUNKS, CHUNK), jnp.int32),
            pltpu.VMEM((HCHUNKS, CHUNK), jnp.int32),
            pltpu.VMEM((CHUNK, D), jnp.float32),
            pltpu.VMEM((CHUNK, D), jnp.float32),
            pltpu.SemaphoreType.DMA,
            pltpu.SemaphoreType.DMA,
            pltpu.VMEM_SHARED((NPAD, D), jnp.float32),
        ],
    )
    def k(y_h, src_h, dst_h, z_h, acc_h, sidx_v, didx_v, rows0_v, rows1_v, gsem0, gsem1, acc_s):
        cid = lax.axis_index("c")
        sid = lax.axis_index("s")
        wid = sid * 2 + cid
        sl = pl.ds(sid * ROWS_PER_TILE, ROWS_PER_TILE)
        pltpu.sync_copy(z_h.at[sl], acc_s.at[sl])
        plsc.subcore_barrier()

        def gather(c, p):
            buf = rows0_v if p == 0 else rows1_v
            sem = gsem0 if p == 0 else gsem1
            pltpu.async_copy(y_h.at[sidx_v.at[c]], buf, sem)

        def gather_wait(c, p):
            buf = rows0_v if p == 0 else rows1_v
            sem = gsem0 if p == 0 else gsem1
            pltpu.make_async_copy(y_h.at[sidx_v.at[c]], buf, sem).wait()

        for ph in range(2):
            base = wid * WCHUNKS + ph * HCHUNKS
            pltpu.sync_copy(src_h.at[pl.ds(base, HCHUNKS)], sidx_v)
            pltpu.sync_copy(dst_h.at[pl.ds(base, HCHUNKS)], didx_v)
            gather(0, 0)

            def body(h, _):
                c0 = h * 2
                gather(c0 + 1, 1)
                gather_wait(c0, 0)
                pltpu.sync_copy(rows0_v, acc_s.at[didx_v.at[c0]], add=True)

                @pl.when(c0 + 2 < HCHUNKS)
                def _():
                    gather(c0 + 2, 0)

                gather_wait(c0 + 1, 1)
                pltpu.sync_copy(rows1_v, acc_s.at[didx_v.at[c0 + 1]],
                                add=True)
                return 0

            lax.fori_loop(0, HCHUNKS // 2, body, 0)
        plsc.subcore_barrier()
        pltpu.sync_copy(acc_s.at[sl], acc_h.at[cid, sl])

    return k(y, src2d, dst2d, zeros128)


def _dinv_block(deg_block):
    d = deg_block[0] + deg_block[1]
    s = d[:, 0:1] + 1.0
    return lax.rsqrt(s)


def _tc1_body(x_r, w_r, deg_r, y_r):
    dinv = _dinv_block(deg_r[...])
    xw = jnp.dot(x_r[...], w_r[...], preferred_element_type=jnp.float32)
    y_r[...] = xw * dinv


def _tc1(x_pad, W1, degp):
    return pl.pallas_call(
        _tc1_body,
        grid=(NBLK,),
        in_specs=[
            pl.BlockSpec((RB, D), lambda i: (i, 0)),
            pl.BlockSpec((D, D), lambda i: (0, 0)),
            pl.BlockSpec((2, RB, 16), lambda i: (0, i, 0)),
        ],
        out_specs=pl.BlockSpec((RB, D), lambda i: (i, 0)),
        out_shape=jax.ShapeDtypeStruct((NPAD, D), jnp.float32),
    )(x_pad, W1, degp)


def _tc2_body(acc_r, y1_r, deg_r, w_r, b_r, y2_r):
    dinv = _dinv_block(deg_r[...])
    acc = acc_r[...]
    s = acc[0] + acc[1] + y1_r[...]
    h = jnp.maximum(s * dinv + b_r[...], 0.0)
    y2_r[...] = jnp.dot(h, w_r[...], preferred_element_type=jnp.float32) * dinv


def _tc2(acc1, y1, degp, W2, b1):
    return pl.pallas_call(
        _tc2_body,
        grid=(NBLK,),
        in_specs=[
            pl.BlockSpec((2, RB, D), lambda i: (0, i, 0)),
            pl.BlockSpec((RB, D), lambda i: (i, 0)),
            pl.BlockSpec((2, RB, 16), lambda i: (0, i, 0)),
            pl.BlockSpec((D, D), lambda i: (0, 0)),
            pl.BlockSpec((1, D), lambda i: (0, 0)),
        ],
        out_specs=pl.BlockSpec((RB, D), lambda i: (i, 0)),
        out_shape=jax.ShapeDtypeStruct((NPAD, D), jnp.float32),
    )(acc1, y1, degp, W2, b1)


def _tc3_body(acc_r, y2_r, deg_r, b2_r, w3_r, b3_r, o_r):
    i = pl.program_id(0)
    dinv = _dinv_block(deg_r[...])
    acc = acc_r[...]
    s = acc[0] + acc[1] + y2_r[...]
    h2 = jnp.maximum(s * dinv + b2_r[...], 0.0)
    t = jnp.dot(h2, w3_r[...], preferred_element_type=jnp.float32)[:, 0:1]
    v = jnp.maximum(t + b3_r[...], 0.0)
    rows = lax.broadcasted_iota(jnp.int32, (RB, 1), 0) + i * RB
    v = jnp.where(rows < N, v, 0.0)
    ps = jnp.sum(v)

    @pl.when(i == 0)
    def _():
        o_r[...] = jnp.zeros_like(o_r)

    o_r[...] = o_r[...] + ps

    @pl.when(i == NBLK - 1)
    def _():
        o_r[...] = o_r[...] * (1.0 / N)


def _tc3(acc2, y2, degp, b2, W3p, b3):
    return pl.pallas_call(
        _tc3_body,
        grid=(NBLK,),
        in_specs=[
            pl.BlockSpec((2, RB, D), lambda i: (0, i, 0)),
            pl.BlockSpec((RB, D), lambda i: (i, 0)),
            pl.BlockSpec((2, RB, 16), lambda i: (0, i, 0)),
            pl.BlockSpec((1, D), lambda i: (0, 0)),
            pl.BlockSpec((D, D), lambda i: (0, 0)),
            pl.BlockSpec((1, 1), lambda i: (0, 0)),
        ],
        out_specs=pl.BlockSpec((1, 1), lambda i: (0, 0)),
        out_shape=jax.ShapeDtypeStruct((1, 1), jnp.float32),
    )(acc2, y2, degp, b2, W3p, b3)


def kernel(x, edge_index, W1, b1, W2, b2, W3, b3):
    f32 = jnp.float32
    ei = edge_index.astype(jnp.int32)
    pad = jnp.full((EPAD - E,), N, jnp.int32)
    src2d = jnp.concatenate([ei[0], pad]).reshape(EPAD // CHUNK, CHUNK)
    dst2d = jnp.concatenate([ei[1], pad]).reshape(EPAD // CHUNK, CHUNK)
    x_pad = jnp.pad(x.astype(f32), ((0, NPAD - N), (0, 0)))
    zeros16 = jnp.zeros((NPAD, 16), f32)
    zeros128 = jnp.zeros((NPAD, D), f32)
    W3p = jnp.pad(W3.astype(f32), ((0, 0), (0, D - W3.shape[1])))

    degp = _sc_deg(dst2d, zeros16)
    y1 = _tc1(x_pad, W1.astype(f32), degp)
    acc1 = _sc_scatter(y1, src2d, dst2d, zeros128)
    y2 = _tc2(acc1, y1, degp, W2.astype(f32), b1.astype(f32).reshape(1, D))
    acc2 = _sc_scatter(y2, src2d, dst2d, zeros128)
    out = _tc3(acc2, y2, degp, b2.astype(f32).reshape(1, D), W3p,
               b3.astype(f32).reshape(1, 1))
    return out
